# baseline (speedup 1.0000x reference)
import functools

import jax
import jax.numpy as jnp
from jax import lax
from jax.experimental import pallas as pl
from jax.experimental.pallas import tpu as pltpu
from jax.experimental.pallas import tpu_sc as plsc

NC = 2
NS = 16
NW = NC * NS
L = 16


@functools.partial(jax.jit, static_argnums=(2, 3))
def _permute_rows(xf, perm, n_rows, d):
    rows_per_w = n_rows // NW
    mesh = plsc.VectorSubcoreMesh(core_axis_name="c", subcore_axis_name="s")

    def body(x_hbm, perm_hbm, out_hbm, perm_v, in_v, out_v):
        wid = lax.axis_index("s") * NC + lax.axis_index("c")
        base = wid * rows_per_w
        pltpu.sync_copy(perm_hbm, perm_v)

        def row(r, carry):
            pltpu.sync_copy(x_hbm.at[base + r], in_v)

            def chunk(j, c):
                idx = perm_v[pl.ds(j * L, L)]
                out_v[pl.ds(j * L, L)] = plsc.load_gather(in_v, [idx])
                return c

            lax.fori_loop(0, d // L, chunk, 0, unroll=8)
            pltpu.sync_copy(out_v, out_hbm.at[base + r])
            return carry

        lax.fori_loop(0, rows_per_w, row, 0)

    fn = pl.kernel(
        body,
        out_type=jax.ShapeDtypeStruct((n_rows, d), jnp.float32),
        mesh=mesh,
        scratch_types=[
            pltpu.VMEM((d,), jnp.int32),
            pltpu.VMEM((d,), jnp.float32),
            pltpu.VMEM((d,), jnp.float32),
        ],
        compiler_params=pltpu.CompilerParams(needs_layout_passes=False),
    )
    return fn(xf, perm)


def kernel(x, perm):
    b, s, d = x.shape
    xf = x.reshape(b * s, d)
    out = _permute_rows(xf, perm, b * s, d)
    return out.reshape(b, s, d)


# speedup vs baseline: 2.5252x; 2.5252x over previous
import functools

import jax
import jax.numpy as jnp
from jax import lax
from jax.experimental import pallas as pl
from jax.experimental.pallas import tpu as pltpu
from jax.experimental.pallas import tpu_sc as plsc

NC = 2
NS = 16
NW = NC * NS
L = 16
R = 4


@functools.partial(jax.jit, static_argnums=(2, 3))
def _permute_rows(xf, perm, n_rows, d):
    rows_per_w = n_rows // NW
    n_chunks = rows_per_w // R
    chunk_e = R * d
    mesh = plsc.VectorSubcoreMesh(core_axis_name="c", subcore_axis_name="s")

    def body(x_hbm, perm_hbm, out_hbm, perm_v,
             in0, in1, ou0, ou1, si0, si1, so0, so1):
        wid = lax.axis_index("s") * NC + lax.axis_index("c")
        base_e = wid * rows_per_w * d
        ins, outs = (in0, in1), (ou0, ou1)
        isems, osems = (si0, si1), (so0, so1)

        pltpu.sync_copy(perm_hbm, perm_v)

        def start_in(c, b):
            pltpu.async_copy(
                x_hbm.at[pl.ds(base_e + c * chunk_e, chunk_e)], ins[b], isems[b])

        def wait_in(b):
            pltpu.make_async_copy(
                x_hbm.at[pl.ds(0, chunk_e)], ins[b], isems[b]).wait()

        def start_out(c, b):
            pltpu.async_copy(
                outs[b], out_hbm.at[pl.ds(base_e + c * chunk_e, chunk_e)],
                osems[b])

        def wait_out(b):
            pltpu.make_async_copy(
                outs[b], out_hbm.at[pl.ds(0, chunk_e)], osems[b]).wait()

        def gather_chunk(b):
            iv, ov = ins[b], outs[b]

            @plsc.parallel_loop(0, d // L, unroll=4)
            def jloop(j):
                idx = perm_v[pl.ds(j * L, L)]
                for r in range(R):
                    ov[pl.ds(r * d + j * L, L)] = plsc.load_gather(
                        iv, [idx + r * d])

        start_in(0, 0)
        start_in(1, 1)
        for b in range(2):
            wait_in(b)
            gather_chunk(b)
            start_out(b, b)
            start_in(b + 2, b)

        @pl.loop(2, n_chunks - 2, step=2)
        def main(g):
            for b in range(2):
                c = g + b
                wait_in(b)
                wait_out(b)
                gather_chunk(b)
                start_out(c, b)
                start_in(c + 2, b)

        for b in range(2):
            wait_in(b)
            wait_out(b)
            gather_chunk(b)
            start_out(n_chunks - 2 + b, b)
        for b in range(2):
            wait_out(b)

    fn = pl.kernel(
        body,
        out_type=jax.ShapeDtypeStruct((n_rows * d,), jnp.float32),
        mesh=mesh,
        scratch_types=[
            pltpu.VMEM((d,), jnp.int32),
            pltpu.VMEM((chunk_e,), jnp.float32),
            pltpu.VMEM((chunk_e,), jnp.float32),
            pltpu.VMEM((chunk_e,), jnp.float32),
            pltpu.VMEM((chunk_e,), jnp.float32),
            pltpu.SemaphoreType.DMA,
            pltpu.SemaphoreType.DMA,
            pltpu.SemaphoreType.DMA,
            pltpu.SemaphoreType.DMA,
        ],
        compiler_params=pltpu.CompilerParams(needs_layout_passes=False),
    )
    return fn(xf, perm)


def kernel(x, perm):
    b, s, d = x.shape
    xf = x.reshape(b * s * d)
    out = _permute_rows(xf, perm, b * s, d)
    return out.reshape(b, s, d)


# speedup vs baseline: 7.3237x; 2.9003x over previous
import functools

import jax
import jax.numpy as jnp
from jax import lax
from jax.experimental import pallas as pl
from jax.experimental.pallas import tpu as pltpu
from jax.experimental.pallas import tpu_sc as plsc

NC = 2
NS = 16
NW = NC * NS
L = 16
R = 4


@functools.partial(jax.jit, static_argnums=(2, 3))
def _permute_rows(x2, perm, n_rows, d):
    rows_per_w = n_rows // NW
    n_chunks = rows_per_w // R
    mesh = plsc.VectorSubcoreMesh(core_axis_name="c", subcore_axis_name="s")

    def body(x_hbm, perm_hbm, out_hbm, perm_v,
             in0, in1, ou0, ou1, si0, si1, so0, so1):
        wid = lax.axis_index("s") * NC + lax.axis_index("c")
        base_r = wid * rows_per_w
        ins, outs = (in0, in1), (ou0, ou1)
        isems, osems = (si0, si1), (so0, so1)

        pltpu.sync_copy(perm_hbm, perm_v)

        def start_in(c, b):
            pltpu.async_copy(
                x_hbm.at[pl.ds(base_r + c * R, R), :], ins[b], isems[b])

        def wait_in(b):
            pltpu.make_async_copy(
                x_hbm.at[pl.ds(0, R), :], ins[b], isems[b]).wait()

        def start_out(c, b):
            pltpu.async_copy(
                outs[b], out_hbm.at[pl.ds(base_r + c * R, R), :], osems[b])

        def wait_out(b):
            pltpu.make_async_copy(
                outs[b], out_hbm.at[pl.ds(0, R), :], osems[b]).wait()

        def gather_chunk(b):
            iv, ov = ins[b], outs[b]

            @plsc.parallel_loop(0, d // L, unroll=4)
            def jloop(j):
                idx = perm_v[pl.ds(j * L, L)]
                for r in range(R):
                    rvec = jnp.full((L,), r, dtype=jnp.int32)
                    ov[r, pl.ds(j * L, L)] = plsc.load_gather(iv, [rvec, idx])

        start_in(0, 0)
        start_in(1, 1)
        for b in range(2):
            wait_in(b)
            gather_chunk(b)
            start_out(b, b)
            start_in(b + 2, b)

        @pl.loop(2, n_chunks - 2, step=2)
        def main(g):
            for b in range(2):
                c = g + b
                wait_in(b)
                wait_out(b)
                gather_chunk(b)
                start_out(c, b)
                start_in(c + 2, b)

        for b in range(2):
            wait_in(b)
            wait_out(b)
            gather_chunk(b)
            start_out(n_chunks - 2 + b, b)
        for b in range(2):
            wait_out(b)

    fn = pl.kernel(
        body,
        out_type=jax.ShapeDtypeStruct((n_rows, d), jnp.float32),
        mesh=mesh,
        scratch_types=[
            pltpu.VMEM((d,), jnp.int32),
            pltpu.VMEM((R, d), jnp.float32),
            pltpu.VMEM((R, d), jnp.float32),
            pltpu.VMEM((R, d), jnp.float32),
            pltpu.VMEM((R, d), jnp.float32),
            pltpu.SemaphoreType.DMA,
            pltpu.SemaphoreType.DMA,
            pltpu.SemaphoreType.DMA,
            pltpu.SemaphoreType.DMA,
        ],
        compiler_params=pltpu.CompilerParams(needs_layout_passes=False),
    )
    return fn(x2, perm)


def kernel(x, perm):
    b, s, d = x.shape
    x2 = x.reshape(b * s, d)
    out = _permute_rows(x2, perm, b * s, d)
    return out.reshape(b, s, d)
